# baseline (speedup 1.0000x reference)
import functools

import jax
import jax.numpy as jnp
from jax import lax
from jax.experimental import pallas as pl
from jax.experimental.pallas import tpu as pltpu
from jax.experimental.pallas import tpu_sc as plsc

N_NODES = 10000
D_IN = 128
D_OUT = 128
N_EDGES = 320000

NC = 2
NS = 16
NW = NC * NS
LANE = 128
FULL_CHUNKS = 78
TAIL_CHUNKS = (N_EDGES - NW * FULL_CHUNKS * LANE) // LANE
GROUP = 13
BINS_PAD = 10240
SLICE = BINS_PAD // NS


def _sc_hist_body(rowa_hbm, rowb_hbm, out_hbm, idx_v, ones_v, buf_v, shared,
                  sem):
    c = lax.axis_index("c")
    s = lax.axis_index("s")
    wid = c * NS + s
    for i in range(LANE // 16):
        ones_v[pl.ds(i * 16, 16)] = jnp.ones((16,), jnp.float32)
    for i in range(SLICE // 16):
        buf_v[pl.ds(i * 16, 16)] = jnp.zeros((16,), jnp.float32)
    pltpu.sync_copy(buf_v, shared.at[pl.ds(s * SLICE, SLICE)])
    pltpu.sync_copy(rowa_hbm.at[wid], idx_v.at[pl.ds(0, FULL_CHUNKS)])

    @pl.when(wid < TAIL_CHUNKS)
    def _():
        pltpu.sync_copy(rowb_hbm.at[wid], idx_v.at[FULL_CHUNKS])

    plsc.subcore_barrier()

    def group(g, carry):
        for jj in range(GROUP):
            pltpu.async_copy(ones_v, shared.at[idx_v.at[g * GROUP + jj]],
                             sem, add=True)
        for jj in range(GROUP):
            pltpu.make_async_copy(ones_v, shared.at[idx_v.at[g * GROUP + jj]],
                                  sem).wait()
        return carry

    lax.fori_loop(0, FULL_CHUNKS // GROUP, group, 0)

    @pl.when(wid < TAIL_CHUNKS)
    def _():
        pltpu.sync_copy(ones_v, shared.at[idx_v.at[FULL_CHUNKS]], add=True)

    plsc.subcore_barrier()
    pltpu.sync_copy(shared.at[pl.ds(s * SLICE, SLICE)], buf_v)
    pltpu.sync_copy(buf_v, out_hbm.at[c, s])


_sc_hist = pl.kernel(
    _sc_hist_body,
    mesh=plsc.VectorSubcoreMesh(core_axis_name="c", subcore_axis_name="s"),
    out_type=jax.ShapeDtypeStruct((NC, NS, SLICE), jnp.float32),
    scratch_types=[
        pltpu.VMEM((FULL_CHUNKS + 1, LANE), jnp.int32),
        pltpu.VMEM((LANE,), jnp.float32),
        pltpu.VMEM((SLICE,), jnp.float32),
        pltpu.VMEM_SHARED((BINS_PAD,), jnp.float32),
        pltpu.SemaphoreType.DMA,
    ],
)


def _tc_body(x_ref, pm_ref, w_ref, c_ref, o_ref):
    t = jnp.dot(x_ref[...], pm_ref[...],
                preferred_element_type=jnp.float32)
    t = jnp.dot(t, w_ref[...],
                preferred_element_type=jnp.float32)
    cnt = c_ref[...]
    total = cnt[:, 0:1] + cnt[:, 1:2]
    o_ref[...] = jnp.where(total > 0.0, t, 0.0)


_BLK = 2000

_tc_matmul = pl.pallas_call(
    _tc_body,
    grid=(N_NODES // _BLK,),
    in_specs=[
        pl.BlockSpec((_BLK, D_IN), lambda i: (i, 0)),
        pl.BlockSpec((D_IN, D_IN), lambda i: (0, 0)),
        pl.BlockSpec((D_IN, D_OUT), lambda i: (0, 0)),
        pl.BlockSpec((_BLK, NC), lambda i: (i, 0)),
    ],
    out_specs=pl.BlockSpec((_BLK, D_OUT), lambda i: (i, 0)),
    out_shape=jax.ShapeDtypeStruct((N_NODES, D_OUT), jnp.float32),
)


def kernel(x, edge_index, perm_matrix, weight):
    row = edge_index[0].astype(jnp.int32)
    n_main = NW * FULL_CHUNKS * LANE
    rowa = row[:n_main].reshape(NW, FULL_CHUNKS, LANE)
    rowb = row[n_main:].reshape(TAIL_CHUNKS, LANE)
    counts = _sc_hist(rowa, rowb)
    cc = counts.reshape(NC, BINS_PAD)[:, :N_NODES].T
    return _tc_matmul(x, perm_matrix, weight, cc)


# speedup vs baseline: 1.0248x; 1.0248x over previous
import functools

import jax
import jax.numpy as jnp
from jax import lax
from jax.experimental import pallas as pl
from jax.experimental.pallas import tpu as pltpu
from jax.experimental.pallas import tpu_sc as plsc

N_NODES = 10000
D_IN = 128
D_OUT = 128
N_EDGES = 320000

NC = 2
NS = 16
NW = NC * NS
LANE = 128
FULL_CHUNKS = 78
TAIL_CHUNKS = (N_EDGES - NW * FULL_CHUNKS * LANE) // LANE
GROUP = 13
BINS_PAD = 10240
SLICE = BINS_PAD // NS


def _sc_hist_body(rowa_hbm, rowb_hbm, out_hbm, idx_v, ones_v, buf_v, shared,
                  sem):
    c = lax.axis_index("c")
    s = lax.axis_index("s")
    wid = c * NS + s
    for i in range(LANE // 16):
        ones_v[pl.ds(i * 16, 16)] = jnp.ones((16,), jnp.float32)
    for i in range(SLICE // 16):
        buf_v[pl.ds(i * 16, 16)] = jnp.zeros((16,), jnp.float32)
    pltpu.sync_copy(buf_v, shared.at[pl.ds(s * SLICE, SLICE)])
    pltpu.sync_copy(rowa_hbm.at[wid], idx_v.at[pl.ds(0, FULL_CHUNKS)])

    @pl.when(wid < TAIL_CHUNKS)
    def _():
        pltpu.sync_copy(rowb_hbm.at[wid], idx_v.at[FULL_CHUNKS])

    plsc.subcore_barrier()

    def group(g, carry):
        for jj in range(GROUP):
            pltpu.async_copy(ones_v, shared.at[idx_v.at[g * GROUP + jj]],
                             sem, add=True)
        for jj in range(GROUP):
            pltpu.make_async_copy(ones_v, shared.at[idx_v.at[g * GROUP + jj]],
                                  sem).wait()
        return carry

    lax.fori_loop(0, FULL_CHUNKS // GROUP, group, 0)

    @pl.when(wid < TAIL_CHUNKS)
    def _():
        pltpu.sync_copy(ones_v, shared.at[idx_v.at[FULL_CHUNKS]], add=True)

    plsc.subcore_barrier()
    pltpu.sync_copy(shared.at[pl.ds(s * SLICE, SLICE)], buf_v)
    pltpu.sync_copy(buf_v, out_hbm.at[c, s])


_sc_hist = pl.kernel(
    _sc_hist_body,
    mesh=plsc.VectorSubcoreMesh(core_axis_name="c", subcore_axis_name="s"),
    out_type=jax.ShapeDtypeStruct((NC, NS, SLICE), jnp.float32),
    scratch_types=[
        pltpu.VMEM((FULL_CHUNKS + 1, LANE), jnp.int32),
        pltpu.VMEM((LANE,), jnp.float32),
        pltpu.VMEM((SLICE,), jnp.float32),
        pltpu.VMEM_SHARED((BINS_PAD,), jnp.float32),
        pltpu.SemaphoreType.DMA,
    ],
)


def _tc_body(x_ref, pm_ref, w_ref, c_ref, o_ref):
    t = jnp.dot(x_ref[...], pm_ref[...],
                preferred_element_type=jnp.float32)
    t = jnp.dot(t, w_ref[...],
                preferred_element_type=jnp.float32)
    cnt = c_ref[...]
    total = cnt[:, 0:1] + cnt[:, 1:2]
    o_ref[...] = jnp.where(total > 0.0, t, 0.0)


_BLK = 5000

_tc_matmul = pl.pallas_call(
    _tc_body,
    grid=(N_NODES // _BLK,),
    in_specs=[
        pl.BlockSpec((_BLK, D_IN), lambda i: (i, 0)),
        pl.BlockSpec((D_IN, D_IN), lambda i: (0, 0)),
        pl.BlockSpec((D_IN, D_OUT), lambda i: (0, 0)),
        pl.BlockSpec((_BLK, NC), lambda i: (i, 0)),
    ],
    out_specs=pl.BlockSpec((_BLK, D_OUT), lambda i: (i, 0)),
    out_shape=jax.ShapeDtypeStruct((N_NODES, D_OUT), jnp.float32),
)


def kernel(x, edge_index, perm_matrix, weight):
    row = edge_index[0].astype(jnp.int32)
    n_main = NW * FULL_CHUNKS * LANE
    rowa = row[:n_main].reshape(NW, FULL_CHUNKS, LANE)
    rowb = row[n_main:].reshape(TAIL_CHUNKS, LANE)
    counts = _sc_hist(rowa, rowb)
    cc = counts.reshape(NC, BINS_PAD)[:, :N_NODES].T
    return _tc_matmul(x, perm_matrix, weight, cc)


# speedup vs baseline: 1.1681x; 1.1398x over previous
import functools

import jax
import jax.numpy as jnp
from jax import lax
from jax.experimental import pallas as pl
from jax.experimental.pallas import tpu as pltpu
from jax.experimental.pallas import tpu_sc as plsc

N_NODES = 10000
D_IN = 128
D_OUT = 128
N_EDGES = 320000

NC = 2
NS = 16
NW = NC * NS
LANE = 128
CHUNKS = 80
E_PER_W = CHUNKS * LANE
E_PAD = NW * E_PER_W
GROUP = 16
BINS_PAD = 10240
SLICE = BINS_PAD // NS


def _sc_hist_body(row_hbm, out_hbm, idx_v, ones_v, buf_v, shared, sem):
    c = lax.axis_index("c")
    s = lax.axis_index("s")
    wid = c * NS + s
    for i in range(LANE // 16):
        ones_v[pl.ds(i * 16, 16)] = jnp.ones((16,), jnp.float32)
    for i in range(SLICE // 16):
        buf_v[pl.ds(i * 16, 16)] = jnp.zeros((16,), jnp.float32)
    pltpu.sync_copy(buf_v, shared.at[pl.ds(s * SLICE, SLICE)])
    pltpu.sync_copy(row_hbm.at[wid], idx_v)
    plsc.subcore_barrier()

    def group(g, carry):
        for jj in range(GROUP):
            pltpu.async_copy(ones_v, shared.at[idx_v.at[g * GROUP + jj]],
                             sem, add=True)
        for jj in range(GROUP):
            pltpu.make_async_copy(ones_v, shared.at[idx_v.at[g * GROUP + jj]],
                                  sem).wait()
        return carry

    lax.fori_loop(0, CHUNKS // GROUP, group, 0)
    plsc.subcore_barrier()
    pltpu.sync_copy(shared.at[pl.ds(s * SLICE, SLICE)], buf_v)
    pltpu.sync_copy(buf_v, out_hbm.at[c, s])


_sc_hist = pl.kernel(
    _sc_hist_body,
    mesh=plsc.VectorSubcoreMesh(core_axis_name="c", subcore_axis_name="s"),
    out_type=jax.ShapeDtypeStruct((NC, NS, SLICE), jnp.float32),
    scratch_types=[
        pltpu.VMEM((CHUNKS, LANE), jnp.int32),
        pltpu.VMEM((LANE,), jnp.float32),
        pltpu.VMEM((SLICE,), jnp.float32),
        pltpu.VMEM_SHARED((BINS_PAD,), jnp.float32),
        pltpu.SemaphoreType.DMA,
    ],
)


def _tc_body(x_ref, pm_ref, w_ref, c_ref, o_ref):
    t = jnp.dot(x_ref[...], pm_ref[...],
                preferred_element_type=jnp.float32)
    t = jnp.dot(t, w_ref[...],
                preferred_element_type=jnp.float32)
    cnt = c_ref[...]
    total = cnt[:, 0:1] + cnt[:, 1:2]
    o_ref[...] = jnp.where(total > 0.0, t, 0.0)


_BLK = 5000

_tc_matmul = pl.pallas_call(
    _tc_body,
    grid=(N_NODES // _BLK,),
    in_specs=[
        pl.BlockSpec((_BLK, D_IN), lambda i: (i, 0)),
        pl.BlockSpec((D_IN, D_IN), lambda i: (0, 0)),
        pl.BlockSpec((D_IN, D_OUT), lambda i: (0, 0)),
        pl.BlockSpec((_BLK, NC), lambda i: (i, 0)),
    ],
    out_specs=pl.BlockSpec((_BLK, D_OUT), lambda i: (i, 0)),
    out_shape=jax.ShapeDtypeStruct((N_NODES, D_OUT), jnp.float32),
)


def kernel(x, edge_index, perm_matrix, weight):
    row = edge_index[0].astype(jnp.int32)
    row_pad = jnp.concatenate(
        [row, jnp.full((E_PAD - N_EDGES,), BINS_PAD - 1, jnp.int32)])
    row3 = row_pad.reshape(NW, CHUNKS, LANE)
    counts = _sc_hist(row3)
    cc = counts.reshape(NC, BINS_PAD)[:, :N_NODES].T
    return _tc_matmul(x, perm_matrix, weight, cc)


# speedup vs baseline: 1.4313x; 1.2253x over previous
import jax
import jax.numpy as jnp
from jax import lax
from jax.experimental import pallas as pl
from jax.experimental.pallas import tpu as pltpu
from jax.experimental.pallas import tpu_sc as plsc

N_NODES = 10000
D_IN = 128
D_OUT = 128
N_EDGES = 320000

NC = 2
NS = 16
NW = NC * NS
LANE = 128
N_CHUNKS = N_EDGES // LANE
BASE_CHUNKS = N_CHUNKS // NW
EXTRA = N_CHUNKS - BASE_CHUNKS * NW
MAX_CHUNKS = BASE_CHUNKS + 1
BINS_PAD = 10240
SLICE = BINS_PAD // NS


def _sc_hist_body(row_hbm, out_hbm, idx_v, ones_v, buf_v, shared, sem):
    c = lax.axis_index("c")
    s = lax.axis_index("s")
    wid = c * NS + s
    n_chunks = BASE_CHUNKS + jnp.where(wid < EXTRA, 1, 0)
    for i in range(LANE // 16):
        ones_v[pl.ds(i * 16, 16)] = jnp.ones((16,), jnp.float32)
    for i in range(SLICE // 16):
        buf_v[pl.ds(i * 16, 16)] = jnp.zeros((16,), jnp.float32)
    pltpu.sync_copy(buf_v, shared.at[pl.ds(s * SLICE, SLICE)])

    def load(k, carry):
        pltpu.async_copy(row_hbm.at[pl.ds((wid + k * NW) * LANE, LANE)],
                         idx_v.at[k], sem)
        return carry

    lax.fori_loop(0, n_chunks, load, 0)

    def load_drain(k, carry):
        pltpu.make_async_copy(row_hbm.at[pl.ds(0, LANE)], idx_v.at[0],
                              sem).wait()
        return carry

    lax.fori_loop(0, n_chunks, load_drain, 0)
    plsc.subcore_barrier()

    def fire(j, carry):
        pltpu.async_copy(ones_v, shared.at[idx_v.at[j]], sem, add=True)
        return carry

    lax.fori_loop(0, n_chunks, fire, 0)

    def drain(j, carry):
        pltpu.make_async_copy(ones_v, shared.at[idx_v.at[j]], sem).wait()
        return carry

    lax.fori_loop(0, n_chunks, drain, 0)
    plsc.subcore_barrier()
    pltpu.sync_copy(shared.at[pl.ds(s * SLICE, SLICE)], buf_v)
    pltpu.sync_copy(buf_v, out_hbm.at[c, s])


_sc_hist = pl.kernel(
    _sc_hist_body,
    mesh=plsc.VectorSubcoreMesh(core_axis_name="c", subcore_axis_name="s"),
    out_type=jax.ShapeDtypeStruct((NC, NS, SLICE), jnp.float32),
    scratch_types=[
        pltpu.VMEM((MAX_CHUNKS, LANE), jnp.int32),
        pltpu.VMEM((LANE,), jnp.float32),
        pltpu.VMEM((SLICE,), jnp.float32),
        pltpu.VMEM_SHARED((BINS_PAD,), jnp.float32),
        pltpu.SemaphoreType.DMA,
    ],
)


def _tc_body(x_ref, pm_ref, w_ref, c_ref, o_ref):
    t = jnp.dot(x_ref[...], pm_ref[...], preferred_element_type=jnp.float32)
    t = jnp.dot(t, w_ref[...], preferred_element_type=jnp.float32)
    cnt = c_ref[...]
    total = cnt[0:1, :] + cnt[1:2, :]
    mask = jnp.transpose(total)
    o_ref[...] = jnp.where(mask > 0.0, t, 0.0)


_BLK = 2048

_tc_matmul = pl.pallas_call(
    _tc_body,
    grid=(pl.cdiv(N_NODES, _BLK),),
    in_specs=[
        pl.BlockSpec((_BLK, D_IN), lambda i: (i, 0)),
        pl.BlockSpec((D_IN, D_IN), lambda i: (0, 0)),
        pl.BlockSpec((D_IN, D_OUT), lambda i: (0, 0)),
        pl.BlockSpec((NC, _BLK), lambda i: (0, i)),
    ],
    out_specs=pl.BlockSpec((_BLK, D_OUT), lambda i: (i, 0)),
    out_shape=jax.ShapeDtypeStruct((N_NODES, D_OUT), jnp.float32),
)


def kernel(x, edge_index, perm_matrix, weight):
    row = edge_index[0].astype(jnp.int32)
    counts = _sc_hist(row)
    cc = counts.reshape(NC, BINS_PAD)
    return _tc_matmul(x, perm_matrix, weight, cc)


# speedup vs baseline: 2.0788x; 1.4524x over previous
import jax
import jax.numpy as jnp
from jax import lax
from jax.experimental import pallas as pl
from jax.experimental.pallas import tpu as pltpu
from jax.experimental.pallas import tpu_sc as plsc

N_NODES = 10000
D_IN = 128
D_OUT = 128
N_EDGES = 320000

NC = 2
NS = 16
NW = NC * NS
LANE = 128
N_CHUNKS = N_EDGES // LANE
BASE_CHUNKS = N_CHUNKS // NW
EXTRA = N_CHUNKS - BASE_CHUNKS * NW
MAX_CHUNKS = BASE_CHUNKS + 1
BINS_PAD = 10240
SLICE = BINS_PAD // NS


def _sc_hist_body(row_hbm, out_hbm, idx_v, ones_v, buf_v, shared, sem):
    c = lax.axis_index("c")
    s = lax.axis_index("s")
    wid = c * NS + s
    n_chunks = BASE_CHUNKS + jnp.where(wid < EXTRA, 1, 0)
    for i in range(LANE // 16):
        ones_v[pl.ds(i * 16, 16)] = jnp.ones((16,), jnp.float32)
    for i in range(SLICE // 16):
        buf_v[pl.ds(i * 16, 16)] = jnp.zeros((16,), jnp.float32)
    pltpu.sync_copy(buf_v, shared.at[pl.ds(s * SLICE, SLICE)])

    def load(k, carry):
        pltpu.async_copy(row_hbm.at[0, pl.ds((wid + k * NW) * LANE, LANE)],
                         idx_v.at[k], sem)
        return carry

    lax.fori_loop(0, n_chunks, load, 0)

    def load_drain(k, carry):
        pltpu.make_async_copy(row_hbm.at[0, pl.ds(0, LANE)], idx_v.at[0],
                              sem).wait()
        return carry

    lax.fori_loop(0, n_chunks, load_drain, 0)
    plsc.subcore_barrier()

    def fire(j, carry):
        pltpu.async_copy(ones_v, shared.at[idx_v.at[j]], sem, add=True)
        return carry

    lax.fori_loop(0, n_chunks, fire, 0)

    def drain(j, carry):
        pltpu.make_async_copy(ones_v, shared.at[idx_v.at[j]], sem).wait()
        return carry

    lax.fori_loop(0, n_chunks, drain, 0)
    plsc.subcore_barrier()
    pltpu.sync_copy(shared.at[pl.ds(s * SLICE, SLICE)], buf_v)
    pltpu.sync_copy(buf_v, out_hbm.at[c, pl.ds(s * SLICE, SLICE)])


_sc_hist = pl.kernel(
    _sc_hist_body,
    mesh=plsc.VectorSubcoreMesh(core_axis_name="c", subcore_axis_name="s"),
    out_type=jax.ShapeDtypeStruct((NC, BINS_PAD), jnp.float32),
    scratch_types=[
        pltpu.VMEM((MAX_CHUNKS, LANE), jnp.int32),
        pltpu.VMEM((LANE,), jnp.float32),
        pltpu.VMEM((SLICE,), jnp.float32),
        pltpu.VMEM_SHARED((BINS_PAD,), jnp.float32),
        pltpu.SemaphoreType.DMA,
    ],
)


def _tc_body(x_ref, pm_ref, w_ref, c_ref, o_ref):
    t = jnp.dot(x_ref[...], pm_ref[...], preferred_element_type=jnp.float32)
    t = jnp.dot(t, w_ref[...], preferred_element_type=jnp.float32)
    cnt = c_ref[...]
    total = cnt[0:1, :] + cnt[1:2, :]
    mask = jnp.transpose(total)
    o_ref[...] = jnp.where(mask > 0.0, t, 0.0)


_BLK = 2048

_tc_matmul = pl.pallas_call(
    _tc_body,
    grid=(pl.cdiv(N_NODES, _BLK),),
    in_specs=[
        pl.BlockSpec((_BLK, D_IN), lambda i: (i, 0)),
        pl.BlockSpec((D_IN, D_IN), lambda i: (0, 0)),
        pl.BlockSpec((D_IN, D_OUT), lambda i: (0, 0)),
        pl.BlockSpec((NC, _BLK), lambda i: (0, i)),
    ],
    out_specs=pl.BlockSpec((_BLK, D_OUT), lambda i: (i, 0)),
    out_shape=jax.ShapeDtypeStruct((N_NODES, D_OUT), jnp.float32),
)


def kernel(x, edge_index, perm_matrix, weight):
    ei = edge_index.astype(jnp.int32)
    cc = _sc_hist(ei)
    return _tc_matmul(x, perm_matrix, weight, cc)
